# Initial kernel scaffold; baseline (speedup 1.0000x reference)
import jax
import jax.numpy as jnp
from jax.experimental import pallas as pl


def kernel(x, embedding, scale):
    raise NotImplementedError("write your pallas kernel here")


# speedup vs baseline: 1.2699x; 1.2699x over previous
import functools

import jax
import jax.numpy as jnp
from jax import lax
from jax.experimental import pallas as pl
from jax.experimental.pallas import tpu as pltpu
from jax.experimental.pallas import tpu_sc as plsc

DIM = 32
CODEBOOK = 8192
BATCH = 16384

TM = 256

NC = 2
NS = 16
L = 16
NW = NC * NS
BPW = BATCH // NW
GCH = 128


def _argmin_body(x16_ref, embT_ref, xn_ref, en_ref, idx_ref):
    xb = x16_ref[...]
    eb = embT_ref[...]
    xn = xn_ref[...]
    en = en_ref[...]
    scores = lax.dot_general(
        xb, eb, (((1,), (0,)), ((), ())),
        preferred_element_type=jnp.float32,
    )
    d = (xn + en) - 2.0 * scores
    m = jnp.min(d, axis=1, keepdims=True)
    ii = lax.broadcasted_iota(jnp.int32, (TM, CODEBOOK), 1)
    idx = jnp.min(jnp.where(d == m, ii, CODEBOOK), axis=1)
    idx_ref[...] = idx.reshape(1, 1, TM)


def _tc_argmin(x16, embT, xn, en):
    grid = BATCH // TM
    return pl.pallas_call(
        _argmin_body,
        grid=(grid,),
        in_specs=[
            pl.BlockSpec((TM, DIM), lambda i: (i, 0)),
            pl.BlockSpec((DIM, CODEBOOK), lambda i: (0, 0)),
            pl.BlockSpec((TM, 1), lambda i: (i, 0)),
            pl.BlockSpec((1, CODEBOOK), lambda i: (0, 0)),
        ],
        out_specs=pl.BlockSpec((1, 1, TM), lambda i: (i, 0, 0)),
        out_shape=jax.ShapeDtypeStruct((grid, 1, TM), jnp.int32),
    )(x16, embT, xn, en)


@functools.partial(
    pl.kernel,
    out_type=(
        jax.ShapeDtypeStruct((BATCH, DIM), jnp.float32),
        jax.ShapeDtypeStruct((NW, L), jnp.float32),
    ),
    mesh=plsc.VectorSubcoreMesh(core_axis_name="c", subcore_axis_name="s"),
    compiler_params=pltpu.CompilerParams(use_tc_tiling_on_sc=False),
    scratch_types=[
        pltpu.VMEM((BPW,), jnp.int32),
        pltpu.VMEM((BPW, DIM), jnp.float32),
        pltpu.VMEM((BPW, DIM), jnp.float32),
        pltpu.VMEM((L,), jnp.float32),
        pltpu.VMEM((L,), jnp.float32),
        pltpu.SemaphoreType.DMA,
    ],
)
def _sc_gather(emb_hbm, idx_hbm, x_hbm, scale_hbm, out_hbm, part_hbm,
               idx_v, rows_v, x_v, scale_v, acc_v, sem):
    wid = lax.axis_index("s") * NC + lax.axis_index("c")
    base = wid * BPW
    pltpu.sync_copy(idx_hbm.at[pl.ds(base, BPW)], idx_v)
    pltpu.sync_copy(x_hbm.at[pl.ds(base, BPW)], x_v)
    pltpu.sync_copy(scale_hbm, scale_v)
    copies = []
    for j in range(BPW // GCH):
        copies.append(pltpu.async_copy(
            emb_hbm.at[idx_v.at[pl.ds(j * GCH, GCH)]],
            rows_v.at[pl.ds(j * GCH, GCH)],
            sem,
        ))
    for c in copies:
        c.wait()

    s = scale_v[...]

    def body(i, acc):
        x0 = x_v[i, pl.ds(0, L)]
        x1 = x_v[i, pl.ds(L, L)]
        t0 = rows_v[i, pl.ds(0, L)] * s
        t1 = rows_v[i, pl.ds(L, L)] * s
        d0 = t0 - x0
        d1 = t1 - x1
        rows_v[i, pl.ds(0, L)] = x0 + d0
        rows_v[i, pl.ds(L, L)] = x1 + d1
        return acc + (d0 * d0 + d1 * d1)

    acc = lax.fori_loop(0, BPW, body, jnp.zeros((L,), jnp.float32))
    acc_v[...] = acc
    pltpu.sync_copy(rows_v, out_hbm.at[pl.ds(base, BPW)])
    pltpu.sync_copy(acc_v, part_hbm.at[wid])


def kernel(x, embedding, scale):
    x16 = x.astype(jnp.bfloat16).astype(jnp.float32)
    embT = embedding.astype(jnp.bfloat16).astype(jnp.float32).T
    xn = jnp.sum(x * x, axis=1, keepdims=True)
    en = jnp.sum(embedding * embedding, axis=1).reshape(1, CODEBOOK)
    idx3 = _tc_argmin(x16, embT, xn, en)
    indices = idx3.reshape(BATCH)
    scale16 = jnp.broadcast_to(scale.astype(jnp.float32), (L,))
    quantized, part = _sc_gather(embedding, indices, x, scale16)
    commit_loss = jnp.sum(part) / (BATCH * DIM)
    return quantized, indices, commit_loss
